# speedup vs baseline: 2.1508x; 2.1508x over previous
import jax
import jax.numpy as jnp
from jax.experimental import pallas as pl

_N = 4096
_H = 8
_A = 32
_F = 256
_HID = 512
_K = 512
_BLK = 256
_A_LOW = -1.0
_A_HIGH = 1.0


def _mlp_block(noise_ref, feat_ref, mu_ref, std_ref, w1f_ref, w1a_ref,
               b1_ref, w2_ref, b2_ref, w3_ref, b3_ref, rew_ref, sumr_ref):
    b = _BLK
    acts = jnp.clip(mu_ref[...] + std_ref[...] * noise_ref[...],
                    _A_LOW, _A_HIGH)
    f = jnp.dot(feat_ref[...], w1f_ref[...],
                preferred_element_type=jnp.float32)
    g = jnp.dot(acts.reshape(b * _H, _A), w1a_ref[...],
                preferred_element_type=jnp.float32)
    h1 = jnp.maximum(g.reshape(b, _H, _HID) + f[:, None, :]
                     + b1_ref[...][None], 0.0)
    h2 = jnp.maximum(jnp.dot(h1.reshape(b * _H, _HID), w2_ref[...],
                             preferred_element_type=jnp.float32)
                     + b2_ref[...], 0.0)
    p = h2 * w3_ref[...]
    q = jnp.sum(p.reshape(b, _H, _HID), axis=2) + b3_ref[...]
    rew_ref[...] = q
    sumr_ref[...] = jnp.sum(q, axis=1, keepdims=True)


def _order_key(x):
    i = jax.lax.bitcast_convert_type(x, jnp.int32)
    return jnp.where(i >= 0, i, (~i) ^ jnp.int32(-2147483648))


def _select_block(sumr2d_ref, sumrcol_ref, noise_ref, mu_ref, std_ref,
                  mu_out, std_out):
    key2 = _order_key(sumr2d_ref[...])

    def bs_body(_, carry):
        lo, hi = carry
        mid = (lo | hi) - ((lo ^ hi) >> 1)
        ge = jnp.sum((key2 >= mid).astype(jnp.int32)) >= _K
        return (jnp.where(ge, mid, lo), jnp.where(ge, hi, mid - 1))
    t, _ = jax.lax.fori_loop(
        0, 34, bs_body, (jnp.int32(-2147483648), jnp.int32(2147483647)))

    cnt_gt = jnp.sum((key2 > t).astype(jnp.int32))
    m = _K - cnt_gt
    row = jax.lax.broadcasted_iota(jnp.int32, (32, 128), 0)
    col = jax.lax.broadcasted_iota(jnp.int32, (32, 128), 1)
    idx2 = row * 128 + col
    eq2 = key2 == t

    def bs2_body(_, carry):
        lo, hi = carry
        mid = (lo & hi) + ((lo ^ hi) >> 1)
        ok = jnp.sum((eq2 & (idx2 <= mid)).astype(jnp.int32)) >= m
        return (jnp.where(ok, lo, mid + 1), jnp.where(ok, mid, hi))
    jcut, _ = jax.lax.fori_loop(0, 13, bs2_body,
                                (jnp.int32(0), jnp.int32(_N - 1)))

    key_col = _order_key(sumrcol_ref[...])
    idx_col = jax.lax.broadcasted_iota(jnp.int32, (_N, 1), 0)
    sel = (key_col > t) | ((key_col == t) & (idx_col <= jcut))
    maskf = sel.astype(jnp.float32)

    acts = jnp.clip(mu_ref[...] + std_ref[...] * noise_ref[...],
                    _A_LOW, _A_HIGH)
    inv_k = jnp.float32(1.0 / _K)
    mu_new = jnp.sum(acts * maskf, axis=0, keepdims=True) * inv_k
    d = (acts - mu_new) * maskf
    var = jnp.sum(d * d, axis=0, keepdims=True) * inv_k
    mu_out[...] = mu_new
    std_out[...] = jnp.maximum(jnp.sqrt(var), 1e-6)


def kernel(noise, feature, mu, std, W1, b1, W2, b2, W3, b3):
    mu3 = mu.reshape(1, _H, _A)
    std3 = std.reshape(1, _H, _A)
    w1f = W1[:_F]
    w1a = W1[_F:]
    b1r = b1.reshape(1, _HID)
    b2r = b2.reshape(1, _HID)
    w3r = W3.reshape(1, _HID)
    b3r = b3.reshape(1, 1)

    rew, sumr = pl.pallas_call(
        _mlp_block,
        grid=(_N // _BLK,),
        in_specs=[
            pl.BlockSpec((_BLK, _H, _A), lambda i: (i, 0, 0)),
            pl.BlockSpec((_BLK, _F), lambda i: (i, 0)),
            pl.BlockSpec((1, _H, _A), lambda i: (0, 0, 0)),
            pl.BlockSpec((1, _H, _A), lambda i: (0, 0, 0)),
            pl.BlockSpec((_F, _HID), lambda i: (0, 0)),
            pl.BlockSpec((_A, _HID), lambda i: (0, 0)),
            pl.BlockSpec((1, _HID), lambda i: (0, 0)),
            pl.BlockSpec((_HID, _HID), lambda i: (0, 0)),
            pl.BlockSpec((1, _HID), lambda i: (0, 0)),
            pl.BlockSpec((1, _HID), lambda i: (0, 0)),
            pl.BlockSpec((1, 1), lambda i: (0, 0)),
        ],
        out_specs=[
            pl.BlockSpec((_BLK, _H), lambda i: (i, 0)),
            pl.BlockSpec((_BLK, 1), lambda i: (i, 0)),
        ],
        out_shape=[
            jax.ShapeDtypeStruct((_N, _H), jnp.float32),
            jax.ShapeDtypeStruct((_N, 1), jnp.float32),
        ],
    )(noise, feature, mu3, std3, w1f, w1a, b1r, W2, b2r, w3r, b3r)

    mu_row = mu.reshape(1, _H * _A)
    std_row = std.reshape(1, _H * _A)
    noise2d = noise.reshape(_N, _H * _A)
    new_mu, new_std = pl.pallas_call(
        _select_block,
        out_shape=[
            jax.ShapeDtypeStruct((1, _H * _A), jnp.float32),
            jax.ShapeDtypeStruct((1, _H * _A), jnp.float32),
        ],
    )(sumr.reshape(32, 128), sumr, noise2d, mu_row, std_row)

    return (rew.reshape(_N, _H, 1), new_mu.reshape(_H, _A),
            new_std.reshape(_H, _A))


# speedup vs baseline: 2.3319x; 1.0842x over previous
import jax
import jax.numpy as jnp
from jax.experimental import pallas as pl
from jax.experimental.pallas import tpu as pltpu

_N = 4096
_H = 8
_A = 32
_F = 256
_HID = 512
_K = 512
_BLK = 256
_NBLK = _N // _BLK
_A_LOW = -1.0
_A_HIGH = 1.0


def _order_key(x):
    i = jax.lax.bitcast_convert_type(x, jnp.int32)
    return jnp.where(i >= 0, i, (~i) ^ jnp.int32(-2147483648))


def _fused(noise_ref, feat_ref, mu_ref, std_ref, w1f_ref, w1a_ref,
           b1_ref, w2_ref, b2_ref, w3_ref, b3_ref, eye_ref,
           rew_ref, mu_out, std_out, acts_s, scol_s, srow_s):
    i = pl.program_id(0)
    b = _BLK
    acts = jnp.clip(mu_ref[...] + std_ref[...] * noise_ref[...],
                    _A_LOW, _A_HIGH)
    f = jnp.dot(feat_ref[...], w1f_ref[...],
                preferred_element_type=jnp.float32)
    g = jnp.dot(acts.reshape(b * _H, _A), w1a_ref[...],
                preferred_element_type=jnp.float32)
    h1 = jnp.maximum(g.reshape(b, _H, _HID) + f[:, None, :]
                     + b1_ref[...][None], 0.0)
    h2 = jnp.maximum(jnp.dot(h1.reshape(b * _H, _HID), w2_ref[...],
                             preferred_element_type=jnp.float32)
                     + b2_ref[...], 0.0)
    p = h2 * w3_ref[...]
    q = jnp.sum(p.reshape(b, _H, _HID), axis=2) + b3_ref[...]
    rew_ref[...] = q

    acts_s[pl.ds(i * b, b), :] = acts.reshape(b, _H * _A)
    scol = jnp.sum(q, axis=1, keepdims=True)
    scol_s[pl.ds(i * b, b), :] = scol
    srow_s[pl.ds(i, 1), :] = jax.lax.dot_general(
        scol, eye_ref[...], (((0,), (0,)), ((), ())),
        preferred_element_type=jnp.float32)

    @pl.when(i == _NBLK - 1)
    def _select():
        key2 = _order_key(srow_s[...])

        def bs_body(_, carry):
            lo, hi = carry
            mid = (lo | hi) - ((lo ^ hi) >> 1)
            ge = jnp.sum((key2 >= mid).astype(jnp.int32)) >= _K
            return (jnp.where(ge, mid, lo), jnp.where(ge, hi, mid - 1))
        t, _ = jax.lax.fori_loop(
            0, 34, bs_body,
            (jnp.int32(-2147483648), jnp.int32(2147483647)))

        cnt_gt = jnp.sum((key2 > t).astype(jnp.int32))
        m = _K - cnt_gt
        row = jax.lax.broadcasted_iota(jnp.int32, (_NBLK, b), 0)
        col = jax.lax.broadcasted_iota(jnp.int32, (_NBLK, b), 1)
        idx2 = row * b + col
        eq2 = key2 == t

        def bs2_body(_, carry):
            lo, hi = carry
            mid = (lo & hi) + ((lo ^ hi) >> 1)
            ok = jnp.sum((eq2 & (idx2 <= mid)).astype(jnp.int32)) >= m
            return (jnp.where(ok, lo, mid + 1), jnp.where(ok, mid, hi))
        jcut, _ = jax.lax.fori_loop(0, 13, bs2_body,
                                    (jnp.int32(0), jnp.int32(_N - 1)))

        key_col = _order_key(scol_s[...])
        idx_col = jax.lax.broadcasted_iota(jnp.int32, (_N, 1), 0)
        sel = (key_col > t) | ((key_col == t) & (idx_col <= jcut))
        maskf = sel.astype(jnp.float32)

        aa = acts_s[...]
        inv_k = jnp.float32(1.0 / _K)
        mu_new = jnp.sum(aa * maskf, axis=0, keepdims=True) * inv_k
        d = (aa - mu_new) * maskf
        var = jnp.sum(d * d, axis=0, keepdims=True) * inv_k
        mu_out[...] = mu_new
        std_out[...] = jnp.maximum(jnp.sqrt(var), 1e-6)


def kernel(noise, feature, mu, std, W1, b1, W2, b2, W3, b3):
    mu3 = mu.reshape(1, _H, _A)
    std3 = std.reshape(1, _H, _A)
    b1r = b1.reshape(1, _HID)
    b2r = b2.reshape(1, _HID)
    w3r = W3.reshape(1, _HID)
    b3r = b3.reshape(1, 1)
    eye = jnp.eye(_BLK, dtype=jnp.float32)

    rew, new_mu, new_std = pl.pallas_call(
        _fused,
        grid=(_NBLK,),
        in_specs=[
            pl.BlockSpec((_BLK, _H, _A), lambda i: (i, 0, 0)),
            pl.BlockSpec((_BLK, _F), lambda i: (i, 0)),
            pl.BlockSpec((1, _H, _A), lambda i: (0, 0, 0)),
            pl.BlockSpec((1, _H, _A), lambda i: (0, 0, 0)),
            pl.BlockSpec((_F, _HID), lambda i: (0, 0)),
            pl.BlockSpec((_A, _HID), lambda i: (_F // _A, 0)),
            pl.BlockSpec((1, _HID), lambda i: (0, 0)),
            pl.BlockSpec((_HID, _HID), lambda i: (0, 0)),
            pl.BlockSpec((1, _HID), lambda i: (0, 0)),
            pl.BlockSpec((1, _HID), lambda i: (0, 0)),
            pl.BlockSpec((1, 1), lambda i: (0, 0)),
            pl.BlockSpec((_BLK, _BLK), lambda i: (0, 0)),
        ],
        out_specs=[
            pl.BlockSpec((_BLK, _H), lambda i: (i, 0)),
            pl.BlockSpec((1, _H * _A), lambda i: (0, 0)),
            pl.BlockSpec((1, _H * _A), lambda i: (0, 0)),
        ],
        out_shape=[
            jax.ShapeDtypeStruct((_N, _H), jnp.float32),
            jax.ShapeDtypeStruct((1, _H * _A), jnp.float32),
            jax.ShapeDtypeStruct((1, _H * _A), jnp.float32),
        ],
        scratch_shapes=[
            pltpu.VMEM((_N, _H * _A), jnp.float32),
            pltpu.VMEM((_N, 1), jnp.float32),
            pltpu.VMEM((_NBLK, _BLK), jnp.float32),
        ],
    )(noise, feature, mu3, std3, W1, W1, b1r, W2, b2r, w3r, b3r, eye)

    return (rew.reshape(_N, _H, 1), new_mu.reshape(_H, _A),
            new_std.reshape(_H, _A))


# speedup vs baseline: 2.3686x; 1.0157x over previous
import jax
import jax.numpy as jnp
from jax.experimental import pallas as pl
from jax.experimental.pallas import tpu as pltpu

_N = 4096
_H = 8
_A = 32
_F = 256
_HID = 512
_K = 512
_BLK = 256
_NBLK = _N // _BLK
_A_LOW = -1.0
_A_HIGH = 1.0


def _order_key(x):
    i = jax.lax.bitcast_convert_type(x, jnp.int32)
    return jnp.where(i >= 0, i, (~i) ^ jnp.int32(-2147483648))


def _fused(noise_ref, feat_ref, mu_ref, std_ref, w1f_ref, w1a_ref,
           b1_ref, w2_ref, b2_ref, w3_ref, b3_ref, eye_ref,
           rew_ref, mu_out, std_out, acts_s, srow_s):
    i = pl.program_id(0)
    b = _BLK
    acts = jnp.clip(mu_ref[...] + std_ref[...] * noise_ref[...],
                    _A_LOW, _A_HIGH)
    f = jnp.dot(feat_ref[...], w1f_ref[...],
                preferred_element_type=jnp.float32)
    g = jnp.dot(acts.reshape(b * _H, _A), w1a_ref[...],
                preferred_element_type=jnp.float32)
    h1 = jnp.maximum(g.reshape(b, _H, _HID) + f[:, None, :]
                     + b1_ref[...][None], 0.0)
    h2 = jnp.maximum(jnp.dot(h1.reshape(b * _H, _HID), w2_ref[...],
                             preferred_element_type=jnp.float32)
                     + b2_ref[...], 0.0)
    p = h2 * w3_ref[...]
    q = jnp.sum(p.reshape(b, _H, _HID), axis=2) + b3_ref[...]
    rew_ref[...] = q

    acts_s[pl.ds(i * b, b), :] = acts.reshape(b, _H * _A)
    srow_s[pl.ds(i, 1), :] = jnp.sum(jnp.transpose(q, (1, 0)),
                                     axis=0, keepdims=True)

    @pl.when(i == _NBLK - 1)
    def _select():
        key2 = _order_key(srow_s[...])

        def bs_body(_, carry):
            lo, hi = carry
            mid = (lo | hi) - ((lo ^ hi) >> 1)
            ge = jnp.sum((key2 >= mid).astype(jnp.int32)) >= _K
            return (jnp.where(ge, mid, lo), jnp.where(ge, hi, mid - 1))
        t, _ = jax.lax.fori_loop(
            0, 34, bs_body,
            (jnp.int32(-2147483648), jnp.int32(2147483647)))

        cnt_gt = jnp.sum((key2 > t).astype(jnp.int32))
        m = _K - cnt_gt
        row = jax.lax.broadcasted_iota(jnp.int32, (_NBLK, b), 0)
        col = jax.lax.broadcasted_iota(jnp.int32, (_NBLK, b), 1)
        idx2 = row * b + col
        eq2 = key2 == t

        def bs2_body(_, carry):
            lo, hi = carry
            mid = (lo & hi) + ((lo ^ hi) >> 1)
            ok = jnp.sum((eq2 & (idx2 <= mid)).astype(jnp.int32)) >= m
            return (jnp.where(ok, lo, mid + 1), jnp.where(ok, mid, hi))
        jcut, _ = jax.lax.fori_loop(0, 13, bs2_body,
                                    (jnp.int32(0), jnp.int32(_N - 1)))

        sel = (key2 > t) | (eq2 & (idx2 <= jcut))
        sel_f = sel.astype(jnp.float32)
        dn = (((1,), (1,)), ((), ()))
        maskf = jnp.concatenate(
            [jax.lax.dot_general(eye_ref[...], sel_f[j:j + 1, :], dn,
                                 preferred_element_type=jnp.float32)
             for j in range(_NBLK)], axis=0)

        aa = acts_s[...]
        inv_k = jnp.float32(1.0 / _K)
        mu_new = jnp.sum(aa * maskf, axis=0, keepdims=True) * inv_k
        d = (aa - mu_new) * maskf
        var = jnp.sum(d * d, axis=0, keepdims=True) * inv_k
        mu_out[...] = mu_new
        std_out[...] = jnp.maximum(jnp.sqrt(var), 1e-6)


def kernel(noise, feature, mu, std, W1, b1, W2, b2, W3, b3):
    mu3 = mu.reshape(1, _H, _A)
    std3 = std.reshape(1, _H, _A)
    b1r = b1.reshape(1, _HID)
    b2r = b2.reshape(1, _HID)
    w3r = W3.reshape(1, _HID)
    b3r = b3.reshape(1, 1)
    eye = jnp.eye(_BLK, dtype=jnp.float32)

    rew, new_mu, new_std = pl.pallas_call(
        _fused,
        grid=(_NBLK,),
        in_specs=[
            pl.BlockSpec((_BLK, _H, _A), lambda i: (i, 0, 0)),
            pl.BlockSpec((_BLK, _F), lambda i: (i, 0)),
            pl.BlockSpec((1, _H, _A), lambda i: (0, 0, 0)),
            pl.BlockSpec((1, _H, _A), lambda i: (0, 0, 0)),
            pl.BlockSpec((_F, _HID), lambda i: (0, 0)),
            pl.BlockSpec((_A, _HID), lambda i: (_F // _A, 0)),
            pl.BlockSpec((1, _HID), lambda i: (0, 0)),
            pl.BlockSpec((_HID, _HID), lambda i: (0, 0)),
            pl.BlockSpec((1, _HID), lambda i: (0, 0)),
            pl.BlockSpec((1, _HID), lambda i: (0, 0)),
            pl.BlockSpec((1, 1), lambda i: (0, 0)),
            pl.BlockSpec((_BLK, _BLK), lambda i: (0, 0)),
        ],
        out_specs=[
            pl.BlockSpec((_BLK, _H), lambda i: (i, 0)),
            pl.BlockSpec((1, _H * _A), lambda i: (0, 0)),
            pl.BlockSpec((1, _H * _A), lambda i: (0, 0)),
        ],
        out_shape=[
            jax.ShapeDtypeStruct((_N, _H), jnp.float32),
            jax.ShapeDtypeStruct((1, _H * _A), jnp.float32),
            jax.ShapeDtypeStruct((1, _H * _A), jnp.float32),
        ],
        scratch_shapes=[
            pltpu.VMEM((_N, _H * _A), jnp.float32),
            pltpu.VMEM((_NBLK, _BLK), jnp.float32),
        ],
    )(noise, feature, mu3, std3, W1, W1, b1r, W2, b2r, w3r, b3r, eye)

    return (rew.reshape(_N, _H, 1), new_mu.reshape(_H, _A),
            new_std.reshape(_H, _A))
